# baseline (speedup 1.0000x reference)
import functools

import jax
import jax.numpy as jnp
from jax import lax
from jax.experimental import pallas as pl
from jax.experimental.pallas import tpu as pltpu
from jax.experimental.pallas import tpu_sc as plsc

B = 16384
F = 26
CAT_VOCAB = 100000
EMB = 32
NUM = 13
DEEP_CAT = F * EMB

NC = 2
NS = 16
NW = NC * NS

ROWS = B * F
RPW = ROWS // NW
SPW = B // NW
CHUNK = 1664
SPC = CHUNK // F
NCHUNK = RPW // CHUNK
WROWS = 1000000 // 16

_sc_mesh = plsc.VectorSubcoreMesh(core_axis_name="c", subcore_axis_name="s")
_sc_params = pltpu.CompilerParams(use_tc_tiling_on_sc=False,
                                  needs_layout_passes=False)


@functools.partial(
    pl.kernel,
    out_type=jax.ShapeDtypeStruct((B,), jnp.float32),
    mesh=_sc_mesh,
    compiler_params=_sc_params,
    scratch_types=[
        pltpu.VMEM((CHUNK,), jnp.int32),
        pltpu.VMEM((CHUNK,), jnp.int32),
        pltpu.VMEM((CHUNK,), jnp.int32),
        pltpu.VMEM((CHUNK, 16), jnp.float32),
        pltpu.VMEM((SPC,), jnp.float32),
        pltpu.SemaphoreType.DMA,
    ],
)
def _sc_wide(widx_hbm, wemb_hbm, wsum_hbm,
             widx_v, whi_v, wlo_v, w16_v, wsum_v, sem_w):
    wid = lax.axis_index("s") * NC + lax.axis_index("c")
    base = wid * RPW
    sbase = wid * SPW
    lane_ids = lax.iota(jnp.int32, 16)

    def body(i, carry):
        off = base + i * CHUNK
        pltpu.sync_copy(widx_hbm.at[pl.ds(off, CHUNK)], widx_v)

        def split_body(k, c):
            w = widx_v[pl.ds(k * 16, 16)]
            whi_v[pl.ds(k * 16, 16)] = lax.shift_right_logical(w, 4)
            wlo_v[pl.ds(k * 16, 16)] = lax.bitwise_and(w, 15)
            return c

        lax.fori_loop(0, CHUNK // 16, split_body, 0)
        pltpu.async_copy(wemb_hbm.at[whi_v], w16_v, sem_w).wait()

        for g in range(SPC // 16):
            def red_body(f, acc):
                row0 = f * SPC + g * 16
                rows = lane_ids + row0
                lanes = wlo_v[pl.ds(row0, 16)]
                return acc + plsc.load_gather(w16_v, [rows, lanes])

            acc = lax.fori_loop(0, F, red_body,
                                jnp.zeros((16,), jnp.float32))
            wsum_v[pl.ds(g * 16, 16)] = acc
        pltpu.sync_copy(wsum_v, wsum_hbm.at[pl.ds(sbase + i * SPC, SPC)])
        return carry

    lax.fori_loop(0, NCHUNK, body, 0)


def _make_deep(nrows):
    rpw = nrows // NW
    nchunk = rpw // CHUNK

    @functools.partial(
        pl.kernel,
        out_type=jax.ShapeDtypeStruct((nrows, EMB), jnp.float32),
        mesh=_sc_mesh,
        compiler_params=_sc_params,
        scratch_types=[
            pltpu.VMEM((CHUNK,), jnp.int32),
            pltpu.VMEM((CHUNK,), jnp.int32),
            pltpu.VMEM((CHUNK, EMB), jnp.float32),
            pltpu.VMEM((CHUNK, EMB), jnp.float32),
            pltpu.SemaphoreType.DMA,
            pltpu.SemaphoreType.DMA,
        ],
    )
    def _sc_deep(didx_hbm, tab_hbm, out_hbm,
                 didx0_v, didx1_v, rows0_v, rows1_v, sem0, sem1):
        wid = lax.axis_index("s") * NC + lax.axis_index("c")
        base = wid * rpw
        idx_bufs = (didx0_v, didx1_v)
        row_bufs = (rows0_v, rows1_v)
        sems = (sem0, sem1)

        pltpu.sync_copy(didx_hbm.at[pl.ds(base, CHUNK)], didx0_v)
        cp = [pltpu.async_copy(tab_hbm.at[didx0_v], rows0_v, sem0), None]
        for i in range(nchunk):
            b = i % 2
            nb = (i + 1) % 2
            if i + 1 < nchunk:
                noff = base + (i + 1) * CHUNK
                pltpu.sync_copy(didx_hbm.at[pl.ds(noff, CHUNK)], idx_bufs[nb])
                cp[b].wait()
                cp[nb] = pltpu.async_copy(tab_hbm.at[idx_bufs[nb]],
                                          row_bufs[nb], sems[nb])
            else:
                cp[b].wait()
            pltpu.sync_copy(row_bufs[b],
                            out_hbm.at[pl.ds(base + i * CHUNK, CHUNK)])

    return _sc_deep


_sc_deep_half = _make_deep(ROWS // 2)


BT = 512


def _mlp_body(g_ref, nx_ref, wl_ref, w1a_ref, w1b_ref, b1_ref, w2_ref,
              b2_ref, w3_ref, b3_ref, w4_ref, b4_ref, out_ref):
    h = jnp.dot(g_ref[...], w1a_ref[...], preferred_element_type=jnp.float32)
    h += jnp.dot(nx_ref[...], w1b_ref[...], preferred_element_type=jnp.float32)
    h = jnp.maximum(h + b1_ref[...], 0.0)
    h = jnp.maximum(
        jnp.dot(h, w2_ref[...], preferred_element_type=jnp.float32) + b2_ref[...], 0.0)
    h = jnp.maximum(
        jnp.dot(h, w3_ref[...], preferred_element_type=jnp.float32) + b3_ref[...], 0.0)
    o = jnp.dot(h, w4_ref[...], preferred_element_type=jnp.float32) + b4_ref[...]
    out_ref[...] = o + wl_ref[...]


def _make_mlp(nb):
    return pl.pallas_call(
        _mlp_body,
        grid=(nb // BT,),
        in_specs=[
        pl.BlockSpec((BT, DEEP_CAT), lambda i: (i, 0)),
        pl.BlockSpec((BT, NUM), lambda i: (i, 0)),
        pl.BlockSpec((BT, 1), lambda i: (i, 0)),
        pl.BlockSpec((DEEP_CAT, 256), lambda i: (0, 0)),
        pl.BlockSpec((NUM, 256), lambda i: (0, 0)),
        pl.BlockSpec((1, 256), lambda i: (0, 0)),
        pl.BlockSpec((256, 128), lambda i: (0, 0)),
        pl.BlockSpec((1, 128), lambda i: (0, 0)),
        pl.BlockSpec((128, 64), lambda i: (0, 0)),
        pl.BlockSpec((1, 64), lambda i: (0, 0)),
            pl.BlockSpec((64, 1), lambda i: (0, 0)),
            pl.BlockSpec((1, 1), lambda i: (0, 0)),
        ],
        out_specs=pl.BlockSpec((BT, 1), lambda i: (i, 0)),
        out_shape=jax.ShapeDtypeStruct((nb, 1), jnp.float32),
    )


_mlp_half = _make_mlp(B // 2)


def kernel(wide_idx, deep_cat_idx, num_x, wide_emb, cat_tables,
           W1, b1, W2, b2, W3, b3, W4, b4):
    tab = cat_tables.reshape(F * CAT_VOCAB, EMB)
    didx = (deep_cat_idx.astype(jnp.int32)
            + (jnp.arange(F, dtype=jnp.int32) * CAT_VOCAB)[None, :]).reshape(ROWS)
    widx = (wide_idx.astype(jnp.int32)
            .reshape(NW, NCHUNK, SPC, F)
            .transpose(0, 1, 3, 2)
            .reshape(ROWS))
    wemb16 = wide_emb.reshape(WROWS, 16)

    wsum = _sc_wide(widx, wemb16)

    hb = B // 2
    hr = ROWS // 2
    outs = []
    for h in range(2):
        gathered = _sc_deep_half(didx[h * hr:(h + 1) * hr], tab)
        outs.append(_mlp_half(
            gathered.reshape(hb, DEEP_CAT),
            num_x[h * hb:(h + 1) * hb],
            wsum[h * hb:(h + 1) * hb].reshape(hb, 1),
            W1[:DEEP_CAT], W1[DEEP_CAT:], b1.reshape(1, 256),
            W2, b2.reshape(1, 128),
            W3, b3.reshape(1, 64),
            W4, b4.reshape(1, 1),
        ))
    return jnp.concatenate(outs, axis=0)


# speedup vs baseline: 1.0023x; 1.0023x over previous
import functools

import jax
import jax.numpy as jnp
from jax import lax
from jax.experimental import pallas as pl
from jax.experimental.pallas import tpu as pltpu
from jax.experimental.pallas import tpu_sc as plsc

B = 16384
F = 26
CAT_VOCAB = 100000
EMB = 32
NUM = 13
DEEP_CAT = F * EMB

NC = 2
NS = 16
NW = NC * NS

ROWS = B * F
RPW = ROWS // NW
SPW = B // NW
CHUNK = 1664
SPC = CHUNK // F
NCHUNK = RPW // CHUNK
WROWS = 1000000 // 16

_sc_mesh = plsc.VectorSubcoreMesh(core_axis_name="c", subcore_axis_name="s")
_sc_params = pltpu.CompilerParams(use_tc_tiling_on_sc=False,
                                  needs_layout_passes=False)


@functools.partial(
    pl.kernel,
    out_type=jax.ShapeDtypeStruct((B,), jnp.float32),
    mesh=_sc_mesh,
    compiler_params=_sc_params,
    scratch_types=[
        pltpu.VMEM((CHUNK,), jnp.int32),
        pltpu.VMEM((CHUNK,), jnp.int32),
        pltpu.VMEM((CHUNK,), jnp.int32),
        pltpu.VMEM((CHUNK, 16), jnp.float32),
        pltpu.VMEM((SPC,), jnp.float32),
        pltpu.SemaphoreType.DMA,
    ],
)
def _sc_wide(widx_hbm, wemb_hbm, wsum_hbm,
             widx_v, whi_v, wlo_v, w16_v, wsum_v, sem_w):
    wid = lax.axis_index("s") * NC + lax.axis_index("c")
    base = wid * RPW
    sbase = wid * SPW
    lane_ids = lax.iota(jnp.int32, 16)

    def body(i, carry):
        off = base + i * CHUNK
        pltpu.sync_copy(widx_hbm.at[pl.ds(off, CHUNK)], widx_v)

        def split_body(k, c):
            w = widx_v[pl.ds(k * 16, 16)]
            whi_v[pl.ds(k * 16, 16)] = lax.shift_right_logical(w, 4)
            wlo_v[pl.ds(k * 16, 16)] = lax.bitwise_and(w, 15)
            return c

        lax.fori_loop(0, CHUNK // 16, split_body, 0)
        pltpu.async_copy(wemb_hbm.at[whi_v], w16_v, sem_w).wait()

        for g in range(SPC // 16):
            def red_body(f, acc):
                row0 = f * SPC + g * 16
                rows = lane_ids + row0
                lanes = wlo_v[pl.ds(row0, 16)]
                return acc + plsc.load_gather(w16_v, [rows, lanes])

            acc = lax.fori_loop(0, F, red_body,
                                jnp.zeros((16,), jnp.float32))
            wsum_v[pl.ds(g * 16, 16)] = acc
        pltpu.sync_copy(wsum_v, wsum_hbm.at[pl.ds(sbase + i * SPC, SPC)])
        return carry

    lax.fori_loop(0, NCHUNK, body, 0)


def _make_deep(nrows):
    rpw = nrows // NW
    nchunk = rpw // CHUNK

    @functools.partial(
        pl.kernel,
        out_type=jax.ShapeDtypeStruct((nrows, EMB), jnp.float32),
        mesh=_sc_mesh,
        compiler_params=_sc_params,
        scratch_types=[
            pltpu.VMEM((CHUNK,), jnp.int32),
            pltpu.VMEM((CHUNK,), jnp.int32),
            pltpu.VMEM((CHUNK, EMB), jnp.float32),
            pltpu.VMEM((CHUNK, EMB), jnp.float32),
            pltpu.SemaphoreType.DMA,
            pltpu.SemaphoreType.DMA,
        ],
    )
    def _sc_deep(didx_hbm, tab_hbm, out_hbm,
                 didx0_v, didx1_v, rows0_v, rows1_v, sem0, sem1):
        wid = lax.axis_index("s") * NC + lax.axis_index("c")
        base = wid * rpw
        idx_bufs = (didx0_v, didx1_v)
        row_bufs = (rows0_v, rows1_v)
        sems = (sem0, sem1)

        pltpu.sync_copy(didx_hbm.at[pl.ds(base, CHUNK)], didx0_v)
        cp = [pltpu.async_copy(tab_hbm.at[didx0_v], rows0_v, sem0), None]
        for i in range(nchunk):
            b = i % 2
            nb = (i + 1) % 2
            if i + 1 < nchunk:
                noff = base + (i + 1) * CHUNK
                pltpu.sync_copy(didx_hbm.at[pl.ds(noff, CHUNK)], idx_bufs[nb])
                cp[b].wait()
                cp[nb] = pltpu.async_copy(tab_hbm.at[idx_bufs[nb]],
                                          row_bufs[nb], sems[nb])
            else:
                cp[b].wait()
            pltpu.sync_copy(row_bufs[b],
                            out_hbm.at[pl.ds(base + i * CHUNK, CHUNK)])

    return _sc_deep


_sc_deep = _make_deep(ROWS)


BT = 512


def _mlp_body(g_ref, nx_ref, wl_ref, w1a_ref, w1b_ref, b1_ref, w2_ref,
              b2_ref, w3_ref, b3_ref, w4_ref, b4_ref, out_ref):
    h = jnp.dot(g_ref[...], w1a_ref[...], preferred_element_type=jnp.float32)
    h += jnp.dot(nx_ref[...], w1b_ref[...], preferred_element_type=jnp.float32)
    h = jnp.maximum(h + b1_ref[...], 0.0)
    h = jnp.maximum(
        jnp.dot(h, w2_ref[...], preferred_element_type=jnp.float32) + b2_ref[...], 0.0)
    h = jnp.maximum(
        jnp.dot(h, w3_ref[...], preferred_element_type=jnp.float32) + b3_ref[...], 0.0)
    o = jnp.dot(h, w4_ref[...], preferred_element_type=jnp.float32) + b4_ref[...]
    out_ref[...] = o + wl_ref[...]


def _make_mlp(nb):
    return pl.pallas_call(
        _mlp_body,
        grid=(nb // BT,),
        in_specs=[
        pl.BlockSpec((BT, DEEP_CAT), lambda i: (i, 0)),
        pl.BlockSpec((BT, NUM), lambda i: (i, 0)),
        pl.BlockSpec((BT, 1), lambda i: (i, 0)),
        pl.BlockSpec((DEEP_CAT, 256), lambda i: (0, 0)),
        pl.BlockSpec((NUM, 256), lambda i: (0, 0)),
        pl.BlockSpec((1, 256), lambda i: (0, 0)),
        pl.BlockSpec((256, 128), lambda i: (0, 0)),
        pl.BlockSpec((1, 128), lambda i: (0, 0)),
        pl.BlockSpec((128, 64), lambda i: (0, 0)),
        pl.BlockSpec((1, 64), lambda i: (0, 0)),
            pl.BlockSpec((64, 1), lambda i: (0, 0)),
            pl.BlockSpec((1, 1), lambda i: (0, 0)),
        ],
        out_specs=pl.BlockSpec((BT, 1), lambda i: (i, 0)),
        out_shape=jax.ShapeDtypeStruct((nb, 1), jnp.float32),
    )


_mlp_call = _make_mlp(B)


def kernel(wide_idx, deep_cat_idx, num_x, wide_emb, cat_tables,
           W1, b1, W2, b2, W3, b3, W4, b4):
    tab = cat_tables.reshape(F * CAT_VOCAB, EMB)
    didx = (deep_cat_idx.astype(jnp.int32)
            + (jnp.arange(F, dtype=jnp.int32) * CAT_VOCAB)[None, :]).reshape(ROWS)
    widx = (wide_idx.astype(jnp.int32)
            .reshape(NW, NCHUNK, SPC, F)
            .transpose(0, 1, 3, 2)
            .reshape(ROWS))
    wemb16 = wide_emb.reshape(WROWS, 16)

    wsum = _sc_wide(widx, wemb16)
    gathered = _sc_deep(didx, tab)

    return _mlp_call(
        gathered.reshape(B, DEEP_CAT), num_x, wsum.reshape(B, 1),
        W1[:DEEP_CAT], W1[DEEP_CAT:], b1.reshape(1, 256),
        W2, b2.reshape(1, 128),
        W3, b3.reshape(1, 64),
        W4, b4.reshape(1, 1),
    )
